# baseline (speedup 1.0000x reference)
import functools

import jax
import jax.numpy as jnp
from jax import lax
from jax.experimental import pallas as pl
from jax.experimental.pallas import tpu as pltpu
from jax.experimental.pallas import tpu_sc as plsc

_LANES = 16


@functools.lru_cache(maxsize=None)
def _build_sc_embed(N, V, D, T, B, NW, C):
    P = T // NW
    NPC = P // C
    mesh = plsc.VectorSubcoreMesh(core_axis_name="c", subcore_axis_name="s")

    @functools.partial(
        pl.kernel,
        mesh=mesh,
        out_type=jax.ShapeDtypeStruct((N, D), jnp.float32),
        scratch_types=[
            pltpu.VMEM((B * P,), jnp.int32),
            pltpu.VMEM((B * C, D), jnp.float32),
            pltpu.VMEM((B * C, D), jnp.float32),
            pltpu.VMEM((C, D), jnp.float32),
            pltpu.VMEM((C, D), jnp.float32),
            pltpu.SemaphoreType.DMA,
            pltpu.SemaphoreType.DMA,
            pltpu.SemaphoreType.DMA,
            pltpu.SemaphoreType.DMA,
            pltpu.SemaphoreType.DMA,
            pltpu.SemaphoreType.DMA,
        ],
    )
    def sc_embed(ids_hbm, tok_hbm, pos_hbm, out_hbm, idx_v,
                 tokb0, tokb1, posb0, posb1,
                 sg0, sg1, st0, st1, sp0, sp1):
        toks = (tokb0, tokb1)
        poss = (posb0, posb1)
        sgs = (sg0, sg1)
        sts = (st0, st1)
        sps = (sp0, sp1)

        wid = lax.axis_index("s") * 2 + lax.axis_index("c")
        pos_base = wid * P

        for b in range(B):
            pltpu.sync_copy(ids_hbm.at[pl.ds(b * T + pos_base, P)],
                            idx_v.at[pl.ds(b * P, P)])

        def start_gathers(pc, side):
            for b in range(B):
                src = tok_hbm.at[idx_v.at[pl.ds(b * P + pc * C, C)]]
                pltpu.async_copy(src, toks[side].at[pl.ds(b * C, C)],
                                 sgs[side])

        def start_pos(pc, side):
            pltpu.async_copy(pos_hbm.at[pl.ds(pos_base + pc * C, C)],
                             poss[side], sps[side])

        def wait_pos(side):
            pltpu.make_async_copy(pos_hbm.at[pl.ds(0, C)], poss[side],
                                  sps[side]).wait()

        def wait_gathers(side):
            pltpu.make_async_copy(pos_hbm.at[pl.ds(0, B * C)], toks[side],
                                  sgs[side]).wait()

        def wait_stores(side):
            pltpu.make_async_copy(toks[side], out_hbm.at[pl.ds(0, B * C)],
                                  sts[side]).wait()

        start_pos(0, 0)
        start_gathers(0, 0)

        def pc_pair(i2, _):
            for side in (0, 1):
                pc = i2 * 2 + side

                @pl.when(jnp.logical_and(pc >= 1, pc + 1 < NPC))
                def _(side=side):
                    wait_stores(side ^ 1)

                @pl.when(pc + 1 < NPC)
                def _(pc=pc, side=side):
                    start_gathers(pc + 1, side ^ 1)
                    start_pos(pc + 1, side ^ 1)

                wait_pos(side)
                wait_gathers(side)

                @plsc.parallel_loop(0, C, 1)
                def _(r, side=side):
                    half = D // _LANES // 2
                    for h in range(2):
                        row = [poss[side][r, pl.ds((h * half + j) * _LANES,
                                                   _LANES)]
                               for j in range(half)]
                        for b in range(B):
                            for j in range(half):
                                sl = pl.ds((h * half + j) * _LANES, _LANES)
                                plsc.addupdate(
                                    toks[side].at[b * C + r, sl], row[j])

                for b in range(B):
                    pltpu.async_copy(
                        toks[side].at[pl.ds(b * C, C)],
                        out_hbm.at[pl.ds(b * T + pos_base + pc * C, C)],
                        sts[side])
            return 0

        lax.fori_loop(0, NPC // 2, pc_pair, 0)
        wait_stores(0)
        wait_stores(1)

    return sc_embed


def kernel(input_ids, token_table, pos_table):
    B, T = input_ids.shape
    V, D = token_table.shape
    N = B * T
    NW = 32
    C = 8
    flat_ids = input_ids.reshape(N).astype(jnp.int32)
    fn = _build_sc_embed(N, V, D, T, B, NW, C)
    out = fn(flat_ids, token_table, pos_table)
    return out.reshape(B, T, D)
